# Initial kernel scaffold; baseline (speedup 1.0000x reference)
import jax
import jax.numpy as jnp
from jax.experimental import pallas as pl


def kernel(h, W_router, W1, W2):
    raise NotImplementedError("write your pallas kernel here")


# speedup vs baseline: 1.4039x; 1.4039x over previous
import functools

import jax
import jax.numpy as jnp
from jax.experimental import pallas as pl
from jax.experimental.pallas import tpu as pltpu

T = 2048
D_MODEL = 1024
E = 8
TOPK = 2
CAPACITY = 512
D_FF = 2048

T_TILE = 512


def _ffn_body(w_ref, h_ref, w1_ref, w2_ref, o_ref):
    e = pl.program_id(1)
    hmid = jax.nn.gelu(
        jnp.dot(h_ref[...], w1_ref[0], preferred_element_type=jnp.float32)
    )
    out = jnp.dot(hmid, w2_ref[0], preferred_element_type=jnp.float32)
    contrib = out * w_ref[0, 0, :][:, None]

    @pl.when(e == 0)
    def _():
        o_ref[...] = contrib

    @pl.when(e > 0)
    def _():
        o_ref[...] += contrib


def kernel(h, W_router, W1, W2):
    Tn, d = h.shape
    logits = h @ W_router
    probs = jax.nn.softmax(logits, axis=-1)
    _, topk_idx = jax.lax.top_k(logits, TOPK)
    mask_te = (
        jnp.zeros((Tn, E), dtype=bool)
        .at[jnp.arange(Tn)[:, None], topk_idx]
        .set(True)
    )
    C = min(CAPACITY, Tn)
    mask_et = mask_te.T
    score_et = jnp.where(mask_et, logits.T, -jnp.inf)
    _, top_indices = jax.lax.top_k(score_et, C)
    kept = (
        jnp.zeros((E, Tn), dtype=bool)
        .at[jnp.arange(E)[:, None], top_indices]
        .set(True)
    ) & mask_et
    w = jnp.where(kept, probs.T, 0.0).reshape(E, 1, Tn)

    n_t = Tn // T_TILE
    y = pl.pallas_call(
        _ffn_body,
        grid=(n_t, E),
        in_specs=[
            pl.BlockSpec((1, 1, T_TILE), lambda t, e: (e, 0, t)),
            pl.BlockSpec((T_TILE, d), lambda t, e: (t, 0)),
            pl.BlockSpec((1, d, D_FF), lambda t, e: (e, 0, 0)),
            pl.BlockSpec((1, D_FF, d), lambda t, e: (e, 0, 0)),
        ],
        out_specs=pl.BlockSpec((T_TILE, d), lambda t, e: (t, 0)),
        out_shape=jax.ShapeDtypeStruct((Tn, d), jnp.float32),
    )(w, h, W1, W2)
    return h + y


# speedup vs baseline: 1.6280x; 1.1596x over previous
import functools

import jax
import jax.numpy as jnp
from jax import lax
from jax.experimental import pallas as pl
from jax.experimental.pallas import tpu as pltpu
from jax.experimental.pallas import tpu_sc as plsc

T = 2048
D_MODEL = 1024
E = 8
TOPK = 2
CAPACITY = 512
D_FF = 2048

NC = 2
NS = 16
NW = NC * NS
B_ROWS = E * CAPACITY
B_PER_W = B_ROWS // NW
CHUNK = 32
N_CHUNKS = B_PER_W // CHUNK


def _sc_gather(table, idx):
    D = table.shape[1]
    idx = idx.reshape(NW, N_CHUNKS, CHUNK)
    mesh = plsc.VectorSubcoreMesh(core_axis_name="c", subcore_axis_name="s")

    @functools.partial(
        pl.kernel,
        mesh=mesh,
        out_type=jax.ShapeDtypeStruct((B_ROWS, D), jnp.float32),
        scratch_types=[
            pltpu.VMEM((N_CHUNKS, CHUNK), jnp.int32),
            pltpu.VMEM((CHUNK, D), jnp.float32),
            pltpu.SemaphoreType.DMA,
        ],
    )
    def k(table_hbm, idx_hbm, out_hbm, idx_v, rows_v, sem):
        wid = lax.axis_index("s") * NC + lax.axis_index("c")
        base = wid * B_PER_W
        pltpu.sync_copy(idx_hbm.at[wid], idx_v)
        for c in range(N_CHUNKS):
            pltpu.async_copy(table_hbm.at[idx_v.at[c]], rows_v, sem).wait()
            pltpu.sync_copy(rows_v, out_hbm.at[pl.ds(base + c * CHUNK, CHUNK)])

    return k(table, idx)


def _ffn_body(x_ref, w1_ref, w2_ref, o_ref):
    hmid = jax.nn.gelu(
        jnp.dot(x_ref[0], w1_ref[0], preferred_element_type=jnp.float32)
    )
    o_ref[0] = jnp.dot(hmid, w2_ref[0], preferred_element_type=jnp.float32)


def _combine_body(h_ref, g_ref, w_ref, o_ref):
    w0 = w_ref[0, 0, :][:, None]
    w1 = w_ref[1, 0, :][:, None]
    o_ref[...] = h_ref[...] + w0 * g_ref[0] + w1 * g_ref[1]


def kernel(h, W_router, W1, W2):
    Tn, d = h.shape
    C = min(CAPACITY, Tn)
    logits = h @ W_router
    probs = jax.nn.softmax(logits, axis=-1)
    _, top2_idx = jax.lax.top_k(logits, TOPK)
    mask_te = (
        jnp.zeros((Tn, E), dtype=bool)
        .at[jnp.arange(Tn)[:, None], top2_idx]
        .set(True)
    )
    mask_et = mask_te.T
    score_et = jnp.where(mask_et, logits.T, -jnp.inf)
    _, slot_tok = jax.lax.top_k(score_et, C)
    e_idx = jnp.arange(E)[:, None]
    valid = jnp.take_along_axis(mask_et, slot_tok, axis=1)
    kept = jnp.zeros((E, Tn), dtype=bool).at[e_idx, slot_tok].set(valid)
    pos = (
        jnp.zeros((E, Tn), dtype=jnp.int32)
        .at[e_idx, slot_tok]
        .set(jnp.broadcast_to(jnp.arange(C, dtype=jnp.int32), (E, C)))
    )
    kept_tk = jnp.take_along_axis(kept.T, top2_idx, axis=1)
    pos_tk = jnp.take_along_axis(pos.T, top2_idx, axis=1)
    s_tk = jnp.where(kept_tk, top2_idx * C + pos_tk, 0)
    p_tk = jnp.take_along_axis(probs, top2_idx, axis=1)
    w_tk = jnp.where(kept_tk, p_tk, 0.0)
    s_km = s_tk.T.reshape(TOPK * Tn).astype(jnp.int32)
    w_km = w_tk.T.reshape(TOPK, 1, Tn)

    xin = _sc_gather(h, slot_tok.reshape(-1).astype(jnp.int32))

    out_flat = pl.pallas_call(
        _ffn_body,
        grid=(E,),
        in_specs=[
            pl.BlockSpec((1, C, d), lambda e: (e, 0, 0)),
            pl.BlockSpec((1, d, D_FF), lambda e: (e, 0, 0)),
            pl.BlockSpec((1, D_FF, d), lambda e: (e, 0, 0)),
        ],
        out_specs=pl.BlockSpec((1, C, d), lambda e: (e, 0, 0)),
        out_shape=jax.ShapeDtypeStruct((E, C, d), jnp.float32),
    )(xin.reshape(E, C, d), W1, W2)

    g = _sc_gather(out_flat.reshape(E * C, d), s_km)

    T_TILE = 512
    n_t = Tn // T_TILE
    y = pl.pallas_call(
        _combine_body,
        grid=(n_t,),
        in_specs=[
            pl.BlockSpec((T_TILE, d), lambda t: (t, 0)),
            pl.BlockSpec((TOPK, T_TILE, d), lambda t: (0, t, 0)),
            pl.BlockSpec((TOPK, 1, T_TILE), lambda t: (0, 0, t)),
        ],
        out_specs=pl.BlockSpec((T_TILE, d), lambda t: (t, 0)),
        out_shape=jax.ShapeDtypeStruct((Tn, d), jnp.float32),
    )(h, g.reshape(TOPK, Tn, d), w_km)
    return y


# speedup vs baseline: 2.6410x; 1.6222x over previous
import functools

import jax
import jax.numpy as jnp
from jax import lax
from jax.experimental import pallas as pl
from jax.experimental.pallas import tpu as pltpu
from jax.experimental.pallas import tpu_sc as plsc

T = 2048
D_MODEL = 1024
E = 8
TOPK = 2
CAPACITY = 512
D_FF = 2048

NC = 2
NS = 16
NW = NC * NS
B_ROWS = TOPK * T
B_PER_W = B_ROWS // NW
CHUNK = 32
N_CHUNKS = B_PER_W // CHUNK
X_PAD_ROWS = E * CAPACITY + CAPACITY
RANK_TILE = 128


def _routing_body(l_ref, lt_ref, sd_ref, sc_ref, w_ref):
    l = l_ref[...]
    lt = lt_ref[...]
    m = jnp.max(l, axis=1, keepdims=True)
    ex = jnp.exp(l - m)
    probs = ex / jnp.sum(ex, axis=1, keepdims=True)
    e1 = jnp.argmax(l, axis=1).astype(jnp.int32)
    iota_te = lax.broadcasted_iota(jnp.int32, (T, E), 1)
    l_no1 = jnp.where(iota_te == e1[:, None], -jnp.inf, l)
    e2 = jnp.argmax(l_no1, axis=1).astype(jnp.int32)
    iota_et = lax.broadcasted_iota(jnp.int32, (E, T), 0)
    sel = (iota_et == e1[None, :]) | (iota_et == e2[None, :])
    s = jnp.where(sel, lt, -jnp.inf)
    rank_cols = []
    iota_tp = lax.broadcasted_iota(jnp.int32, (T, RANK_TILE), 0)
    iota_tt = lax.broadcasted_iota(jnp.int32, (T, RANK_TILE), 1)
    for j in range(T // RANK_TILE):
        st = s[:, j * RANK_TILE : (j + 1) * RANK_TILE]
        a = s[:, :, None]
        b = st[:, None, :]
        beats = jnp.where(
            (a > b)
            | ((a == b) & (iota_tp < (iota_tt + j * RANK_TILE))[None, :, :]),
            1.0,
            0.0,
        )
        rank_cols.append(jnp.sum(beats, axis=1))
    rank = jnp.concatenate(rank_cols, axis=1)
    kept = sel & (rank < float(CAPACITY))
    keptf = kept.astype(jnp.float32)
    iota_r = lax.broadcasted_iota(jnp.int32, (T, T), 0)
    iota_c = lax.broadcasted_iota(jnp.int32, (T, T), 1)
    tri = jnp.where(iota_r <= iota_c, 1.0, 0.0)
    incl = jnp.dot(keptf, tri, preferred_element_type=jnp.float32)
    pos = (incl - keptf).astype(jnp.int32)

    def pick_et(arr, e_sel):
        return jnp.sum(
            jnp.where(iota_et == e_sel[None, :], arr, 0), axis=0
        )

    def pick_te(arr, e_sel):
        return jnp.sum(
            jnp.where(iota_te == e_sel[:, None], arr, 0), axis=1
        )

    for k, e_k in enumerate((e1, e2)):
        kept_k = pick_et(keptf, e_k) > 0.0
        pos_k = pick_et(pos, e_k)
        slot_k = e_k * CAPACITY + pos_k
        w_k = jnp.where(kept_k, pick_te(probs, e_k), 0.0)
        sd_ref[k, :] = jnp.where(kept_k, slot_k, E * CAPACITY)
        sc_ref[k, :] = jnp.where(kept_k, slot_k, 0)
        w_ref[k, :] = w_k


def _sc_scatter_rows(src, idx):
    D = src.shape[1]
    idx = idx.reshape(NW, N_CHUNKS, CHUNK)
    mesh = plsc.VectorSubcoreMesh(core_axis_name="c", subcore_axis_name="s")

    @functools.partial(
        pl.kernel,
        mesh=mesh,
        out_type=jax.ShapeDtypeStruct((X_PAD_ROWS, D), jnp.float32),
        scratch_types=[
            pltpu.VMEM((N_CHUNKS, CHUNK), jnp.int32),
            pltpu.VMEM((CHUNK, D), jnp.float32),
            pltpu.SemaphoreType.DMA,
        ],
    )
    def k(src_hbm, idx_hbm, out_hbm, idx_v, rows_v, sem):
        wid = lax.axis_index("s") * NC + lax.axis_index("c")
        tok_base = (wid % (T // B_PER_W)) * B_PER_W
        pltpu.sync_copy(idx_hbm.at[wid], idx_v)
        for c in range(N_CHUNKS):
            pltpu.sync_copy(src_hbm.at[pl.ds(tok_base + c * CHUNK, CHUNK)], rows_v)
            pltpu.async_copy(rows_v, out_hbm.at[idx_v.at[c]], sem).wait()

    return k(src, idx)


def _sc_gather(table, idx):
    D = table.shape[1]
    idx = idx.reshape(NW, N_CHUNKS, CHUNK)
    mesh = plsc.VectorSubcoreMesh(core_axis_name="c", subcore_axis_name="s")

    @functools.partial(
        pl.kernel,
        mesh=mesh,
        out_type=jax.ShapeDtypeStruct((B_ROWS, D), jnp.float32),
        scratch_types=[
            pltpu.VMEM((N_CHUNKS, CHUNK), jnp.int32),
            pltpu.VMEM((CHUNK, D), jnp.float32),
            pltpu.SemaphoreType.DMA,
        ],
    )
    def k(table_hbm, idx_hbm, out_hbm, idx_v, rows_v, sem):
        wid = lax.axis_index("s") * NC + lax.axis_index("c")
        base = wid * B_PER_W
        pltpu.sync_copy(idx_hbm.at[wid], idx_v)
        for c in range(N_CHUNKS):
            pltpu.async_copy(table_hbm.at[idx_v.at[c]], rows_v, sem).wait()
            pltpu.sync_copy(rows_v, out_hbm.at[pl.ds(base + c * CHUNK, CHUNK)])

    return k(table, idx)


def _ffn_body(x_ref, w1_ref, w2_ref, o_ref):
    hmid = jax.nn.gelu(
        jnp.dot(x_ref[...], w1_ref[0], preferred_element_type=jnp.float32)
    )
    o_ref[...] = jnp.dot(hmid, w2_ref[0], preferred_element_type=jnp.float32)


def _combine_body(h_ref, g_ref, w_ref, o_ref):
    w0 = w_ref[0, 0, :][:, None]
    w1 = w_ref[1, 0, :][:, None]
    g0 = jnp.where(w0 > 0.0, w0 * g_ref[0], 0.0)
    g1 = jnp.where(w1 > 0.0, w1 * g_ref[1], 0.0)
    o_ref[...] = h_ref[...] + g0 + g1


def kernel(h, W_router, W1, W2):
    Tn, d = h.shape
    C = min(CAPACITY, Tn)
    logits = h @ W_router
    logits_t = logits.T

    s_disp, s_comb, w_km = pl.pallas_call(
        _routing_body,
        out_shape=[
            jax.ShapeDtypeStruct((TOPK, Tn), jnp.int32),
            jax.ShapeDtypeStruct((TOPK, Tn), jnp.int32),
            jax.ShapeDtypeStruct((TOPK, Tn), jnp.float32),
        ],
    )(logits, logits_t)

    xin = _sc_scatter_rows(h, s_disp.reshape(-1))

    out_flat = pl.pallas_call(
        _ffn_body,
        grid=(E,),
        in_specs=[
            pl.BlockSpec((C, d), lambda e: (e, 0)),
            pl.BlockSpec((1, d, D_FF), lambda e: (e, 0, 0)),
            pl.BlockSpec((1, D_FF, d), lambda e: (e, 0, 0)),
        ],
        out_specs=pl.BlockSpec((C, d), lambda e: (e, 0)),
        out_shape=jax.ShapeDtypeStruct((E * C, d), jnp.float32),
    )(xin, W1, W2)

    g = _sc_gather(out_flat, s_comb.reshape(-1))

    T_TILE = 512
    n_t = Tn // T_TILE
    y = pl.pallas_call(
        _combine_body,
        grid=(n_t,),
        in_specs=[
            pl.BlockSpec((T_TILE, d), lambda t: (t, 0)),
            pl.BlockSpec((TOPK, T_TILE, d), lambda t: (0, t, 0)),
            pl.BlockSpec((TOPK, 1, T_TILE), lambda t: (0, 0, t)),
        ],
        out_specs=pl.BlockSpec((T_TILE, d), lambda t: (t, 0)),
        out_shape=jax.ShapeDtypeStruct((Tn, d), jnp.float32),
    )(h, g.reshape(TOPK, Tn, d), w_km.reshape(TOPK, 1, Tn))
    return y


# speedup vs baseline: 2.9810x; 1.1287x over previous
import functools

import jax
import jax.numpy as jnp
from jax import lax
from jax.experimental import pallas as pl
from jax.experimental.pallas import tpu as pltpu
from jax.experimental.pallas import tpu_sc as plsc

T = 2048
D_MODEL = 1024
E = 8
TOPK = 2
CAPACITY = 512
D_FF = 2048

NC = 2
NS = 16
NW = NC * NS
B_ROWS = TOPK * T
B_PER_W = B_ROWS // NW
CHUNK = 32
N_CHUNKS = B_PER_W // CHUNK
X_PAD_ROWS = E * CAPACITY + CAPACITY
RANK_TILE = 128


def _routing_body(l_ref, lt_ref, sd_ref, sc_ref, w_ref):
    l = l_ref[...]
    lt = lt_ref[...]
    m = jnp.max(l, axis=1, keepdims=True)
    ex = jnp.exp(l - m)
    probs = ex / jnp.sum(ex, axis=1, keepdims=True)
    e1 = jnp.argmax(l, axis=1).astype(jnp.int32)
    iota_te = lax.broadcasted_iota(jnp.int32, (T, E), 1)
    l_no1 = jnp.where(iota_te == e1[:, None], -jnp.inf, l)
    e2 = jnp.argmax(l_no1, axis=1).astype(jnp.int32)
    iota_et = lax.broadcasted_iota(jnp.int32, (E, T), 0)
    sel = (iota_et == e1[None, :]) | (iota_et == e2[None, :])
    s = jnp.where(sel, lt, -jnp.inf)
    int_min = jnp.int32(-2147483648)
    bits = lax.bitcast_convert_type(s, jnp.int32)
    key = bits ^ ((bits >> 31) & jnp.int32(2147483647))
    key = jnp.where(sel, key, int_min)
    lo = jnp.full((E, 1), int_min, jnp.int32)
    hi = jnp.full((E, 1), 2147483647, jnp.int32)
    cap = jnp.float32(CAPACITY)
    for _ in range(32):
        mid = (lo >> 1) + (hi >> 1) + (lo & hi & 1)
        cnt = jnp.sum((key > mid).astype(jnp.float32), axis=1, keepdims=True)
        pred = cnt >= cap
        lo = jnp.where(pred, mid + 1, lo)
        hi = jnp.where(pred, hi, mid)
    thr = lo
    gt = key > thr
    n_gt = jnp.sum(gt.astype(jnp.float32), axis=1, keepdims=True)
    need = cap - n_gt
    eq = (key == thr) & sel
    eqf = eq.astype(jnp.float32)
    iota_r = lax.broadcasted_iota(jnp.int32, (T, T), 0)
    iota_c = lax.broadcasted_iota(jnp.int32, (T, T), 1)
    tri = jnp.where(iota_r <= iota_c, 1.0, 0.0)
    eq_incl = jnp.dot(eqf, tri, preferred_element_type=jnp.float32)
    kept = gt | (eq & ((eq_incl - eqf) < need))
    keptf = kept.astype(jnp.float32)
    incl = jnp.dot(keptf, tri, preferred_element_type=jnp.float32)
    pos = (incl - keptf).astype(jnp.int32)

    def pick_et(arr, e_sel):
        return jnp.sum(
            jnp.where(iota_et == e_sel[None, :], arr, 0), axis=0
        )

    def pick_te(arr, e_sel):
        return jnp.sum(
            jnp.where(iota_te == e_sel[:, None], arr, 0), axis=1
        )

    for k, e_k in enumerate((e1, e2)):
        kept_k = pick_et(keptf, e_k) > 0.0
        pos_k = pick_et(pos, e_k)
        slot_k = e_k * CAPACITY + pos_k
        w_k = jnp.where(kept_k, pick_te(probs, e_k), 0.0)
        sd_ref[k, :] = jnp.where(kept_k, slot_k, E * CAPACITY)
        sc_ref[k, :] = jnp.where(kept_k, slot_k, 0)
        w_ref[k, :] = w_k


def _sc_scatter_rows(src, idx):
    D = src.shape[1]
    idx = idx.reshape(NW, N_CHUNKS, CHUNK)
    mesh = plsc.VectorSubcoreMesh(core_axis_name="c", subcore_axis_name="s")

    @functools.partial(
        pl.kernel,
        mesh=mesh,
        out_type=jax.ShapeDtypeStruct((X_PAD_ROWS, D), jnp.float32),
        scratch_types=[
            pltpu.VMEM((N_CHUNKS, CHUNK), jnp.int32),
            pltpu.VMEM((2, CHUNK, D), jnp.float32),
            pltpu.SemaphoreType.DMA,
            pltpu.SemaphoreType.DMA,
            pltpu.SemaphoreType.DMA,
            pltpu.SemaphoreType.DMA,
        ],
    )
    def k(src_hbm, idx_hbm, out_hbm, idx_v, rows_v, s0, s1, s2, s3):
        wid = lax.axis_index("s") * NC + lax.axis_index("c")
        tok_base = (wid % (T // B_PER_W)) * B_PER_W
        rsem = (s0, s1)
        wsem = (s2, s3)
        pltpu.sync_copy(idx_hbm.at[wid], idx_v)

        def read(c):
            return pltpu.async_copy(
                src_hbm.at[pl.ds(tok_base + c * CHUNK, CHUNK)],
                rows_v.at[c % 2],
                rsem[c % 2],
            )

        rp = [None] * N_CHUNKS
        rp[0] = read(0)
        rp[1] = read(1)
        for c in range(N_CHUNKS):
            rp[c].wait()
            wp = pltpu.async_copy(
                rows_v.at[c % 2], out_hbm.at[idx_v.at[c]], wsem[c % 2]
            )
            wp.wait()
            if c + 2 < N_CHUNKS:
                rp[c + 2] = read(c + 2)

    return k(src, idx)


def _sc_gather(table, idx):
    D = table.shape[1]
    idx = idx.reshape(NW, N_CHUNKS, CHUNK)
    mesh = plsc.VectorSubcoreMesh(core_axis_name="c", subcore_axis_name="s")

    @functools.partial(
        pl.kernel,
        mesh=mesh,
        out_type=jax.ShapeDtypeStruct((B_ROWS, D), jnp.float32),
        scratch_types=[
            pltpu.VMEM((N_CHUNKS, CHUNK), jnp.int32),
            pltpu.VMEM((2, CHUNK, D), jnp.float32),
            pltpu.SemaphoreType.DMA,
            pltpu.SemaphoreType.DMA,
            pltpu.SemaphoreType.DMA,
            pltpu.SemaphoreType.DMA,
        ],
    )
    def k(table_hbm, idx_hbm, out_hbm, idx_v, rows_v, s0, s1, s2, s3):
        wid = lax.axis_index("s") * NC + lax.axis_index("c")
        base = wid * B_PER_W
        gsem = (s0, s1)
        wsem = (s2, s3)
        pltpu.sync_copy(idx_hbm.at[wid], idx_v)

        def gather(c):
            return pltpu.async_copy(
                table_hbm.at[idx_v.at[c]], rows_v.at[c % 2], gsem[c % 2]
            )

        gp = [None] * N_CHUNKS
        gp[0] = gather(0)
        gp[1] = gather(1)
        for c in range(N_CHUNKS):
            gp[c].wait()
            wp = pltpu.async_copy(
                rows_v.at[c % 2],
                out_hbm.at[pl.ds(base + c * CHUNK, CHUNK)],
                wsem[c % 2],
            )
            wp.wait()
            if c + 2 < N_CHUNKS:
                gp[c + 2] = gather(c + 2)

    return k(table, idx)


def _ffn_body(x_ref, w1_ref, w2_ref, o_ref):
    hmid = jax.nn.gelu(
        jnp.dot(x_ref[...], w1_ref[0], preferred_element_type=jnp.float32)
    )
    o_ref[...] = jnp.dot(hmid, w2_ref[0], preferred_element_type=jnp.float32)


def _combine_body(h_ref, g_ref, w_ref, o_ref):
    w0 = w_ref[0, 0, :][:, None]
    w1 = w_ref[1, 0, :][:, None]
    g0 = jnp.where(w0 > 0.0, w0 * g_ref[0], 0.0)
    g1 = jnp.where(w1 > 0.0, w1 * g_ref[1], 0.0)
    o_ref[...] = h_ref[...] + g0 + g1


def kernel(h, W_router, W1, W2):
    Tn, d = h.shape
    C = min(CAPACITY, Tn)
    logits = h @ W_router
    logits_t = logits.T

    s_disp, s_comb, w_km = pl.pallas_call(
        _routing_body,
        out_shape=[
            jax.ShapeDtypeStruct((TOPK, Tn), jnp.int32),
            jax.ShapeDtypeStruct((TOPK, Tn), jnp.int32),
            jax.ShapeDtypeStruct((TOPK, Tn), jnp.float32),
        ],
    )(logits, logits_t)

    xin = _sc_scatter_rows(h, s_disp.reshape(-1))

    out_flat = pl.pallas_call(
        _ffn_body,
        grid=(E,),
        in_specs=[
            pl.BlockSpec((C, d), lambda e: (e, 0)),
            pl.BlockSpec((1, d, D_FF), lambda e: (e, 0, 0)),
            pl.BlockSpec((1, D_FF, d), lambda e: (e, 0, 0)),
        ],
        out_specs=pl.BlockSpec((C, d), lambda e: (e, 0)),
        out_shape=jax.ShapeDtypeStruct((E * C, d), jnp.float32),
    )(xin, W1, W2)

    g = _sc_gather(out_flat, s_comb.reshape(-1))

    T_TILE = 512
    n_t = Tn // T_TILE
    y = pl.pallas_call(
        _combine_body,
        grid=(n_t,),
        in_specs=[
            pl.BlockSpec((T_TILE, d), lambda t: (t, 0)),
            pl.BlockSpec((TOPK, T_TILE, d), lambda t: (0, t, 0)),
            pl.BlockSpec((TOPK, 1, T_TILE), lambda t: (0, 0, t)),
        ],
        out_specs=pl.BlockSpec((T_TILE, d), lambda t: (t, 0)),
        out_shape=jax.ShapeDtypeStruct((Tn, d), jnp.float32),
    )(h, g.reshape(TOPK, Tn, d), w_km.reshape(TOPK, 1, Tn))
    return y
